# Initial kernel scaffold; baseline (speedup 1.0000x reference)
import jax
import jax.numpy as jnp
from jax.experimental import pallas as pl


def kernel(x, edge_weight, u1_W0, u1_b0, u1_p, u1_W1, u1_b1, u1_Wu, u1_bu, bn1_g, bn1_b, u2_W0, u2_b0, u2_p, u2_W1, u2_b1, u2_Wu, u2_bu, bn2_g, bn2_b, lin_W, lin_b, edge_index):
    raise NotImplementedError("write your pallas kernel here")


# speedup vs baseline: 1.3219x; 1.3219x over previous
import functools
import math

import jax
import jax.numpy as jnp
from jax import lax
from jax.experimental import pallas as pl

N = 10000
H = 64
BR = 1000
GRID_N = N // BR
KP = 5000
GRID_K = KP // BR

_f32 = jnp.float32


def _mm_scale_body(a_ref, w_ref, s_ref, o_ref):
    o_ref[...] = s_ref[...] * jnp.dot(a_ref[...], w_ref[...],
                                      preferred_element_type=_f32)


def _mm_scale(a, w, s, grid):
    m, k = a.shape
    br = m // grid
    return pl.pallas_call(
        _mm_scale_body,
        grid=(grid,),
        in_specs=[
            pl.BlockSpec((br, k), lambda i: (i, 0)),
            pl.BlockSpec((k, w.shape[1]), lambda i: (0, 0)),
            pl.BlockSpec((br, 1), lambda i: (i, 0)),
        ],
        out_specs=pl.BlockSpec((br, w.shape[1]), lambda i: (i, 0)),
        out_shape=jax.ShapeDtypeStruct((m, w.shape[1]), _f32),
    )(a, w, s)


def _add_mm_scale_body(r_ref, u_ref, w_ref, s_ref, o_ref):
    o_ref[...] = s_ref[...] * jnp.dot(r_ref[...] + u_ref[...], w_ref[...],
                                      preferred_element_type=_f32)


def _add_mm_scale(r, u, w, s):
    m, k = r.shape
    br = m // GRID_N
    return pl.pallas_call(
        _add_mm_scale_body,
        grid=(GRID_N,),
        in_specs=[
            pl.BlockSpec((br, k), lambda i: (i, 0)),
            pl.BlockSpec((br, k), lambda i: (i, 0)),
            pl.BlockSpec((k, w.shape[1]), lambda i: (0, 0)),
            pl.BlockSpec((br, 1), lambda i: (i, 0)),
        ],
        out_specs=pl.BlockSpec((br, w.shape[1]), lambda i: (i, 0)),
        out_shape=jax.ShapeDtypeStruct((m, w.shape[1]), _f32),
    )(r, u, w, s)


def _gcn_post_score_body(acc_ref, xs_ref, d_ref, b_ref, p_ref, x1_ref, sc_ref):
    x1 = jnp.maximum(d_ref[...] * (acc_ref[...] + xs_ref[...]) + b_ref[...],
                     0.0)
    x1_ref[...] = x1
    sc_ref[...] = jnp.tanh(jnp.dot(x1, p_ref[...],
                                   preferred_element_type=_f32))


def _gcn_post_score(acc, xs, dinv, b, p_hat):
    return pl.pallas_call(
        _gcn_post_score_body,
        grid=(GRID_N,),
        in_specs=[
            pl.BlockSpec((BR, H), lambda i: (i, 0)),
            pl.BlockSpec((BR, H), lambda i: (i, 0)),
            pl.BlockSpec((BR, 1), lambda i: (i, 0)),
            pl.BlockSpec((1, H), lambda i: (0, 0)),
            pl.BlockSpec((H, 1), lambda i: (0, 0)),
        ],
        out_specs=[
            pl.BlockSpec((BR, H), lambda i: (i, 0)),
            pl.BlockSpec((BR, 1), lambda i: (i, 0)),
        ],
        out_shape=[
            jax.ShapeDtypeStruct((N, H), _f32),
            jax.ShapeDtypeStruct((N, 1), _f32),
        ],
    )(acc, xs, dinv, b, p_hat)


def _pool_post_body(z_ref, u1_ref, u2_ref, d2_ref, dp_ref, b_ref, o_ref):
    t = (2.0 - d2_ref[...]) * z_ref[...] + 2.0 * u1_ref[...] + u2_ref[...]
    o_ref[...] = jnp.maximum(dp_ref[...] * t + b_ref[...], 0.0)


def _pool_post(z, u1p, u2p, d2p, dinvp, b):
    return pl.pallas_call(
        _pool_post_body,
        grid=(GRID_K,),
        in_specs=[
            pl.BlockSpec((BR, H), lambda i: (i, 0)),
            pl.BlockSpec((BR, H), lambda i: (i, 0)),
            pl.BlockSpec((BR, H), lambda i: (i, 0)),
            pl.BlockSpec((BR, 1), lambda i: (i, 0)),
            pl.BlockSpec((BR, 1), lambda i: (i, 0)),
            pl.BlockSpec((1, H), lambda i: (0, 0)),
        ],
        out_specs=pl.BlockSpec((BR, H), lambda i: (i, 0)),
        out_shape=jax.ShapeDtypeStruct((KP, H), _f32),
    )(z, u1p, u2p, d2p, dinvp, b)


def _gcn_post_bn_body(acc_ref, xs_ref, d_ref, b_ref, g2_ref, b2_ref, o_ref):
    h = jnp.maximum(d_ref[...] * (acc_ref[...] + xs_ref[...]) + b_ref[...],
                    0.0)
    o_ref[...] = h * g2_ref[...] + b2_ref[...]


def _gcn_post_bn(acc, xs, dinv, b, g2, b2):
    return pl.pallas_call(
        _gcn_post_bn_body,
        grid=(GRID_N,),
        in_specs=[
            pl.BlockSpec((BR, H), lambda i: (i, 0)),
            pl.BlockSpec((BR, H), lambda i: (i, 0)),
            pl.BlockSpec((BR, 1), lambda i: (i, 0)),
            pl.BlockSpec((1, H), lambda i: (0, 0)),
            pl.BlockSpec((1, H), lambda i: (0, 0)),
            pl.BlockSpec((1, H), lambda i: (0, 0)),
        ],
        out_specs=pl.BlockSpec((BR, H), lambda i: (i, 0)),
        out_shape=jax.ShapeDtypeStruct((N, H), _f32),
    )(acc, xs, dinv, b, g2, b2)


def _final_body(a_ref, w_ref, b_ref, o_ref):
    o_ref[...] = jnp.dot(a_ref[...], w_ref[...],
                         preferred_element_type=_f32) + b_ref[...]


def _final_linear(a, w, b):
    return pl.pallas_call(
        _final_body,
        grid=(GRID_N,),
        in_specs=[
            pl.BlockSpec((BR, H), lambda i: (i, 0)),
            pl.BlockSpec((H, w.shape[1]), lambda i: (0, 0)),
            pl.BlockSpec((1, w.shape[1]), lambda i: (0, 0)),
        ],
        out_specs=pl.BlockSpec((BR, w.shape[1]), lambda i: (i, 0)),
        out_shape=jax.ShapeDtypeStruct((N, w.shape[1]), _f32),
    )(a, w, b)


def _edge_apply(X, row, col, w):
    return jnp.zeros_like(X).at[col].add(w[:, None] * X[row])


def _edge_apply_vec(m, row, col, w):
    return jnp.zeros_like(m).at[col].add(w * m[row])


def kernel(x, edge_weight, u1_W0, u1_b0, u1_p, u1_W1, u1_b1, u1_Wu, u1_bu,
           bn1_g, bn1_b, u2_W0, u2_b0, u2_p, u2_W1, u2_b1, u2_Wu, u2_bu,
           bn2_g, bn2_b, lin_W, lin_b, edge_index):
    n = x.shape[0]
    row = edge_index[0].astype(jnp.int32)
    col = edge_index[1].astype(jnp.int32)
    w = edge_weight
    E = w.shape[0]

    deg = jnp.zeros(n, _f32).at[col].add(w) + 1.0
    dinv = lax.rsqrt(deg)[:, None]

    key = row * n + col
    rkey = col * n + row
    order = jnp.argsort(key)
    sk = key[order]
    sw = w[order]
    uid = jnp.cumsum(jnp.concatenate([jnp.zeros(1, jnp.int32),
                                      (sk[1:] != sk[:-1]).astype(jnp.int32)]))
    gw = jnp.zeros(E, _f32).at[uid].add(sw)
    lo = jnp.searchsorted(sk, rkey, side='left')
    lo_c = jnp.clip(lo, 0, E - 1)
    match = (lo < E) & (sk[lo_c] == rkey)
    revw = jnp.where(match, gw[uid[lo_c]], 0.0)
    selfw = jnp.zeros(n, _f32).at[row].add(jnp.where(row == col, w, 0.0))
    D2 = jnp.zeros(n, _f32).at[row].add(w * revw) + 2.0 * selfw + 1.0

    def gcn_front(xin, W):
        xs = _mm_scale(xin, W, dinv, GRID_N)
        acc = _edge_apply(xs, row, col, w)
        return xs, acc

    def unet(xin, W0, b0, p, W1, b1, Wu, bu, g2, b2):
        xs, acc = gcn_front(xin, W0)
        p_hat = (p / jnp.linalg.norm(p))[:, None]
        x1, score = _gcn_post_score(acc, xs, dinv, b0[None, :], p_hat)
        vals, perm = lax.top_k(score[:, 0], KP)

        m = jnp.zeros(n, _f32).at[perm].set(1.0)
        Sm = _edge_apply_vec(m, row, col, w)
        SSm = _edge_apply_vec(Sm, row, col, w)
        q = m + 2.0 * Sm + SSm
        degp = q[perm] - D2[perm] + 1.0
        dinvp = jnp.where(degp > 0, lax.rsqrt(jnp.maximum(degp, 1e-12)),
                          0.0)[:, None]

        zscale = dinvp * vals[:, None]
        z = _mm_scale(x1[perm], W1, zscale, GRID_K)
        Z = jnp.zeros((n, H), _f32).at[perm].set(z)
        U1 = _edge_apply(Z, row, col, w)
        U2 = _edge_apply(U1, row, col, w)
        xp2 = _pool_post(z, U1[perm], U2[perm], D2[perm][:, None], dinvp,
                         b1[None, :])
        UP = jnp.zeros((n, H), _f32).at[perm].set(xp2)

        xs_u = _add_mm_scale(x1, UP, Wu, dinv)
        acc_u = _edge_apply(xs_u, row, col, w)
        return _gcn_post_bn(acc_u, xs_u, dinv, bu[None, :], g2[None, :],
                            b2[None, :])

    g2_1 = bn1_g / jnp.sqrt(1.0 + 1e-5)
    g2_2 = bn2_g / jnp.sqrt(1.0 + 1e-5)
    h = unet(x, u1_W0, u1_b0, u1_p, u1_W1, u1_b1, u1_Wu, u1_bu, g2_1, bn1_b)
    h = unet(h, u2_W0, u2_b0, u2_p, u2_W1, u2_b1, u2_Wu, u2_bu, g2_2, bn2_b)
    return _final_linear(h, lin_W, lin_b[None, :])


# speedup vs baseline: 2.8522x; 2.1576x over previous
import functools
import math

import jax
import jax.numpy as jnp
from jax import lax
from jax.experimental import pallas as pl
from jax.experimental.pallas import tpu as pltpu
from jax.experimental.pallas import tpu_sc as plsc

N = 10000
H = 64
BR = 1000
GRID_N = N // BR
KP = 5000
GRID_K = KP // BR

_f32 = jnp.float32


def _mm_scale_body(a_ref, w_ref, s_ref, o_ref):
    o_ref[...] = s_ref[...] * jnp.dot(a_ref[...], w_ref[...],
                                      preferred_element_type=_f32)


def _mm_scale(a, w, s, grid):
    m, k = a.shape
    br = m // grid
    return pl.pallas_call(
        _mm_scale_body,
        grid=(grid,),
        in_specs=[
            pl.BlockSpec((br, k), lambda i: (i, 0)),
            pl.BlockSpec((k, w.shape[1]), lambda i: (0, 0)),
            pl.BlockSpec((br, 1), lambda i: (i, 0)),
        ],
        out_specs=pl.BlockSpec((br, w.shape[1]), lambda i: (i, 0)),
        out_shape=jax.ShapeDtypeStruct((m, w.shape[1]), _f32),
    )(a, w, s)


def _add_mm_scale_body(r_ref, u_ref, w_ref, s_ref, o_ref):
    o_ref[...] = s_ref[...] * jnp.dot(r_ref[...] + u_ref[...], w_ref[...],
                                      preferred_element_type=_f32)


def _add_mm_scale(r, u, w, s):
    m, k = r.shape
    br = m // GRID_N
    return pl.pallas_call(
        _add_mm_scale_body,
        grid=(GRID_N,),
        in_specs=[
            pl.BlockSpec((br, k), lambda i: (i, 0)),
            pl.BlockSpec((br, k), lambda i: (i, 0)),
            pl.BlockSpec((k, w.shape[1]), lambda i: (0, 0)),
            pl.BlockSpec((br, 1), lambda i: (i, 0)),
        ],
        out_specs=pl.BlockSpec((br, w.shape[1]), lambda i: (i, 0)),
        out_shape=jax.ShapeDtypeStruct((m, w.shape[1]), _f32),
    )(r, u, w, s)


def _gcn_post_score_body(acc_ref, xs_ref, d_ref, b_ref, p_ref, x1_ref, sc_ref):
    x1 = jnp.maximum(d_ref[...] * (acc_ref[...] + xs_ref[...]) + b_ref[...],
                     0.0)
    x1_ref[...] = x1
    sc_ref[...] = jnp.tanh(jnp.dot(x1, p_ref[...],
                                   preferred_element_type=_f32))


def _gcn_post_score(acc, xs, dinv, b, p_hat):
    return pl.pallas_call(
        _gcn_post_score_body,
        grid=(GRID_N,),
        in_specs=[
            pl.BlockSpec((BR, H), lambda i: (i, 0)),
            pl.BlockSpec((BR, H), lambda i: (i, 0)),
            pl.BlockSpec((BR, 1), lambda i: (i, 0)),
            pl.BlockSpec((1, H), lambda i: (0, 0)),
            pl.BlockSpec((H, 1), lambda i: (0, 0)),
        ],
        out_specs=[
            pl.BlockSpec((BR, H), lambda i: (i, 0)),
            pl.BlockSpec((BR, 1), lambda i: (i, 0)),
        ],
        out_shape=[
            jax.ShapeDtypeStruct((N, H), _f32),
            jax.ShapeDtypeStruct((N, 1), _f32),
        ],
    )(acc, xs, dinv, b, p_hat)


def _pool_post_body(z_ref, u1_ref, u2_ref, d2_ref, dp_ref, b_ref, o_ref):
    t = (2.0 - d2_ref[...]) * z_ref[...] + 2.0 * u1_ref[...] + u2_ref[...]
    o_ref[...] = jnp.maximum(dp_ref[...] * t + b_ref[...], 0.0)


def _pool_post(z, u1p, u2p, d2p, dinvp, b):
    return pl.pallas_call(
        _pool_post_body,
        grid=(GRID_K,),
        in_specs=[
            pl.BlockSpec((BR, H), lambda i: (i, 0)),
            pl.BlockSpec((BR, H), lambda i: (i, 0)),
            pl.BlockSpec((BR, H), lambda i: (i, 0)),
            pl.BlockSpec((BR, 1), lambda i: (i, 0)),
            pl.BlockSpec((BR, 1), lambda i: (i, 0)),
            pl.BlockSpec((1, H), lambda i: (0, 0)),
        ],
        out_specs=pl.BlockSpec((BR, H), lambda i: (i, 0)),
        out_shape=jax.ShapeDtypeStruct((KP, H), _f32),
    )(z, u1p, u2p, d2p, dinvp, b)


def _gcn_post_bn_body(acc_ref, xs_ref, d_ref, b_ref, g2_ref, b2_ref, o_ref):
    h = jnp.maximum(d_ref[...] * (acc_ref[...] + xs_ref[...]) + b_ref[...],
                    0.0)
    o_ref[...] = h * g2_ref[...] + b2_ref[...]


def _gcn_post_bn(acc, xs, dinv, b, g2, b2):
    return pl.pallas_call(
        _gcn_post_bn_body,
        grid=(GRID_N,),
        in_specs=[
            pl.BlockSpec((BR, H), lambda i: (i, 0)),
            pl.BlockSpec((BR, H), lambda i: (i, 0)),
            pl.BlockSpec((BR, 1), lambda i: (i, 0)),
            pl.BlockSpec((1, H), lambda i: (0, 0)),
            pl.BlockSpec((1, H), lambda i: (0, 0)),
            pl.BlockSpec((1, H), lambda i: (0, 0)),
        ],
        out_specs=pl.BlockSpec((BR, H), lambda i: (i, 0)),
        out_shape=jax.ShapeDtypeStruct((N, H), _f32),
    )(acc, xs, dinv, b, g2, b2)


def _final_body(a_ref, w_ref, b_ref, o_ref):
    o_ref[...] = jnp.dot(a_ref[...], w_ref[...],
                         preferred_element_type=_f32) + b_ref[...]


def _final_linear(a, w, b):
    return pl.pallas_call(
        _final_body,
        grid=(GRID_N,),
        in_specs=[
            pl.BlockSpec((BR, H), lambda i: (i, 0)),
            pl.BlockSpec((H, w.shape[1]), lambda i: (0, 0)),
            pl.BlockSpec((1, w.shape[1]), lambda i: (0, 0)),
        ],
        out_specs=pl.BlockSpec((BR, w.shape[1]), lambda i: (i, 0)),
        out_shape=jax.ShapeDtypeStruct((N, w.shape[1]), _f32),
    )(a, w, b)


NTILES = 16
NCORES = 2
EPT = 5120
NCHUNK = 40
CHUNK = 128
E_PAD = NCORES * NTILES * EPT
N_PAD = 10240
ROWS_PER_TILE = N_PAD // NTILES


def _make_sc_edge_apply(width):
    lanes = 16
    nvec = width // lanes
    mesh = plsc.VectorSubcoreMesh(core_axis_name="c", subcore_axis_name="s")

    @functools.partial(
        pl.kernel,
        mesh=mesh,
        compiler_params=pltpu.CompilerParams(use_tc_tiling_on_sc=False),
        out_type=jax.ShapeDtypeStruct((NCORES, N_PAD, width), _f32),
        scratch_types=[
            pltpu.VMEM((CHUNK,), jnp.int32),
            pltpu.VMEM((CHUNK,), jnp.int32),
            pltpu.VMEM((CHUNK, lanes), _f32),
            pltpu.VMEM((CHUNK, width), _f32),
            pltpu.VMEM_SHARED((N_PAD, width), _f32),
            pltpu.SemaphoreType.DMA,
        ],
    )
    def apply_k(x_hbm, er_hbm, ec_hbm, ew_hbm, z_hbm, out_hbm,
                ridx, cidx, wv, rows, acc, sem):
        c = lax.axis_index("c")
        s = lax.axis_index("s")
        rs = pl.ds(s * ROWS_PER_TILE, ROWS_PER_TILE)
        pltpu.sync_copy(z_hbm.at[rs], acc.at[rs])
        plsc.subcore_barrier()

        def chunk(j, carry):
            pltpu.sync_copy(er_hbm.at[c, s, j], ridx)
            pltpu.sync_copy(ec_hbm.at[c, s, j], cidx)
            pltpu.sync_copy(ew_hbm.at[c, s, j], wv)
            pltpu.async_copy(x_hbm.at[ridx], rows, sem).wait()

            def scale16(g, cc):
                base = g * 16
                for l in range(16):
                    wrow = wv[base + l, :]
                    for f in range(nvec):
                        sl = pl.ds(f * lanes, lanes)
                        rows[base + l, sl] = rows[base + l, sl] * wrow
                return cc

            lax.fori_loop(0, CHUNK // 16, scale16, 0)
            pltpu.sync_copy(rows, acc.at[cidx], add=True)
            return carry

        lax.fori_loop(0, NCHUNK, chunk, 0)
        plsc.subcore_barrier()
        pltpu.sync_copy(acc.at[rs], out_hbm.at[c, rs])

    return apply_k


_sc_apply_64 = _make_sc_edge_apply(64)
_sc_apply_16 = _make_sc_edge_apply(16)


KP_PAD = 5120
RG_PER_TILE = KP_PAD // (NCORES * NTILES)
RG_CHUNK = 80


def _make_sc_row_gather(width):
    mesh = plsc.VectorSubcoreMesh(core_axis_name="c", subcore_axis_name="s")

    @functools.partial(
        pl.kernel,
        mesh=mesh,
        compiler_params=pltpu.CompilerParams(use_tc_tiling_on_sc=False),
        out_type=jax.ShapeDtypeStruct((KP_PAD, width), _f32),
        scratch_types=[
            pltpu.VMEM((2, RG_CHUNK), jnp.int32),
            pltpu.VMEM((RG_CHUNK, width), _f32),
            pltpu.SemaphoreType.DMA,
        ],
    )
    def gather_k(x_hbm, idx_hbm, out_hbm, idxv, rows, sem):
        c = lax.axis_index("c")
        s = lax.axis_index("s")
        tid = c * NTILES + s
        pltpu.sync_copy(idx_hbm.at[tid], idxv)
        for j in range(2):
            pltpu.async_copy(x_hbm.at[idxv.at[j]], rows, sem).wait()
            pltpu.sync_copy(
                rows, out_hbm.at[pl.ds(tid * RG_PER_TILE + j * RG_CHUNK,
                                       RG_CHUNK)])

    return gather_k


_sc_gather_64 = _make_sc_row_gather(64)


def _rows_gather(X, perm_pad):
    Xp = jnp.pad(X, ((0, N_PAD - N), (0, 0)))
    return _sc_gather_64(Xp, perm_pad)[:KP]


def _make_sc_row_scatter(width):
    mesh = plsc.VectorSubcoreMesh(core_axis_name="c", subcore_axis_name="s")

    @functools.partial(
        pl.kernel,
        mesh=mesh,
        compiler_params=pltpu.CompilerParams(use_tc_tiling_on_sc=False),
        out_type=jax.ShapeDtypeStruct((NCORES, N_PAD, width), _f32),
        scratch_types=[
            pltpu.VMEM((2, RG_CHUNK), jnp.int32),
            pltpu.VMEM((RG_CHUNK, width), _f32),
            pltpu.VMEM_SHARED((N_PAD, width), _f32),
            pltpu.SemaphoreType.DMA,
        ],
    )
    def scatter_k(z_hbm, idx_hbm, zero_hbm, out_hbm, idxv, rows, acc, sem):
        c = lax.axis_index("c")
        s = lax.axis_index("s")
        tid = c * NTILES + s
        rs = pl.ds(s * ROWS_PER_TILE, ROWS_PER_TILE)
        pltpu.sync_copy(zero_hbm.at[rs], acc.at[rs])
        pltpu.sync_copy(idx_hbm.at[tid], idxv)
        plsc.subcore_barrier()
        for j in range(2):
            base = tid * RG_PER_TILE + j * RG_CHUNK
            pltpu.sync_copy(z_hbm.at[pl.ds(base, RG_CHUNK)], rows)
            pltpu.sync_copy(rows, acc.at[idxv.at[j]], add=True)
        plsc.subcore_barrier()
        pltpu.sync_copy(acc.at[rs], out_hbm.at[c, rs])

    return scatter_k


_sc_scatter_64 = _make_sc_row_scatter(64)


def _rows_scatter(z, perm_pad, zeros64):
    parts = _sc_scatter_64(z, perm_pad, zeros64)
    return (parts[0] + parts[1])[:N]


def _prep_edges(row, col, w):
    E = row.shape[0]
    pad = E_PAD - E
    row_p = jnp.pad(row, (0, pad)).reshape(NCORES, NTILES, NCHUNK, CHUNK)
    col_p = jnp.pad(col, (0, pad)).reshape(NCORES, NTILES, NCHUNK, CHUNK)
    w_p = jnp.pad(w, (0, pad))
    w16 = jnp.broadcast_to(
        w_p[:, None], (E_PAD, 16)).reshape(NCORES, NTILES, NCHUNK, CHUNK, 16)
    return row_p, col_p, w16


def _edge_apply(X, ed, zeros64):
    Xp = jnp.pad(X, ((0, N_PAD - N), (0, 0)))
    parts = _sc_apply_64(Xp, ed[0], ed[1], ed[2], zeros64)
    return (parts[0] + parts[1])[:N]


def _edge_apply_16(X, ed, zeros16):
    Xp = jnp.pad(X, ((0, N_PAD - N), (0, 0)))
    parts = _sc_apply_16(Xp, ed[0], ed[1], ed[2], zeros16)
    return (parts[0] + parts[1])[:N]


def kernel(x, edge_weight, u1_W0, u1_b0, u1_p, u1_W1, u1_b1, u1_Wu, u1_bu,
           bn1_g, bn1_b, u2_W0, u2_b0, u2_p, u2_W1, u2_b1, u2_Wu, u2_bu,
           bn2_g, bn2_b, lin_W, lin_b, edge_index):
    n = x.shape[0]
    row = edge_index[0].astype(jnp.int32)
    col = edge_index[1].astype(jnp.int32)
    w = edge_weight
    E = w.shape[0]

    ed = _prep_edges(row, col, w)
    zeros64 = jnp.zeros((N_PAD, 64), _f32)
    zeros16 = jnp.zeros((N_PAD, 16), _f32)
    ones16 = jnp.ones((n, 16), _f32)

    deg = _edge_apply_16(ones16, ed, zeros16)[:, 0] + 1.0
    dinv = lax.rsqrt(deg)[:, None]

    key = row * n + col
    rkey = col * n + row
    order = jnp.argsort(key)
    sk = key[order]
    sw = w[order]
    uid = jnp.cumsum(jnp.concatenate([jnp.zeros(1, jnp.int32),
                                      (sk[1:] != sk[:-1]).astype(jnp.int32)]))
    gw = jnp.zeros(E, _f32).at[uid].add(sw)
    lo = jnp.searchsorted(sk, rkey, side='left')
    lo_c = jnp.clip(lo, 0, E - 1)
    match = (lo < E) & (sk[lo_c] == rkey)
    revw = jnp.where(match, gw[uid[lo_c]], 0.0)
    wd2 = w * revw + 2.0 * jnp.where(row == col, w, 0.0)
    ed_d2 = _prep_edges(col, row, wd2)
    D2 = _edge_apply_16(ones16, ed_d2, zeros16)[:, 0] + 1.0

    def unet(xin, W0, b0, p, W1, b1, Wu, bu, g2, b2):
        xs = _mm_scale(xin, W0, dinv, GRID_N)
        acc = _edge_apply(xs, ed, zeros64)
        p_hat = (p / jnp.linalg.norm(p))[:, None]
        x1, score = _gcn_post_score(acc, xs, dinv, b0[None, :], p_hat)
        vals, perm = lax.top_k(score[:, 0], KP)
        perm_pad = jnp.pad(perm, (0, KP_PAD - KP)).reshape(
            NCORES * NTILES, 2, RG_CHUNK)

        mask = score[:, 0] >= vals[KP - 1]
        m16 = jnp.where(mask, 1.0, 0.0)[:, None] * jnp.ones((1, 16), _f32)
        Sm = _edge_apply_16(m16, ed, zeros16)
        SSm = _edge_apply_16(Sm, ed, zeros16)
        q = m16[:, 0] + 2.0 * Sm[:, 0] + SSm[:, 0]
        degp = q[perm] - D2[perm] + 1.0
        dinvp = jnp.where(degp > 0, lax.rsqrt(jnp.maximum(degp, 1e-12)),
                          0.0)[:, None]

        zscale = dinvp * vals[:, None]
        z = _mm_scale(_rows_gather(x1, perm_pad), W1, zscale, GRID_K)
        z_pad = jnp.pad(z, ((0, KP_PAD - KP), (0, 0)))
        Z = _rows_scatter(z_pad, perm_pad, zeros64)
        U1 = _edge_apply(Z, ed, zeros64)
        U2 = _edge_apply(U1, ed, zeros64)
        xp2 = _pool_post(z, _rows_gather(U1, perm_pad),
                         _rows_gather(U2, perm_pad), D2[perm][:, None],
                         dinvp, b1[None, :])
        UP = _rows_scatter(jnp.pad(xp2, ((0, KP_PAD - KP), (0, 0))),
                           perm_pad, zeros64)

        xs_u = _add_mm_scale(x1, UP, Wu, dinv)
        acc_u = _edge_apply(xs_u, ed, zeros64)
        return _gcn_post_bn(acc_u, xs_u, dinv, bu[None, :], g2[None, :],
                            b2[None, :])

    g2_1 = bn1_g / jnp.sqrt(1.0 + 1e-5)
    g2_2 = bn2_g / jnp.sqrt(1.0 + 1e-5)
    h = unet(x, u1_W0, u1_b0, u1_p, u1_W1, u1_b1, u1_Wu, u1_bu, g2_1, bn1_b)
    h = unet(h, u2_W0, u2_b0, u2_p, u2_W1, u2_b1, u2_Wu, u2_bu, g2_2, bn2_b)
    return _final_linear(h, lin_W, lin_b[None, :])


# speedup vs baseline: 3.0238x; 1.0602x over previous
import functools
import math

import jax
import jax.numpy as jnp
from jax import lax
from jax.experimental import pallas as pl
from jax.experimental.pallas import tpu as pltpu
from jax.experimental.pallas import tpu_sc as plsc

N = 10000
H = 64
BR = 1000
GRID_N = N // BR
KP = 5000
GRID_K = KP // BR

_f32 = jnp.float32


def _mm_scale_body(a_ref, w_ref, s_ref, o_ref):
    o_ref[...] = s_ref[...] * jnp.dot(a_ref[...], w_ref[...],
                                      preferred_element_type=_f32)


def _mm_scale(a, w, s, grid):
    m, k = a.shape
    br = m // grid
    return pl.pallas_call(
        _mm_scale_body,
        grid=(grid,),
        in_specs=[
            pl.BlockSpec((br, k), lambda i: (i, 0)),
            pl.BlockSpec((k, w.shape[1]), lambda i: (0, 0)),
            pl.BlockSpec((br, 1), lambda i: (i, 0)),
        ],
        out_specs=pl.BlockSpec((br, w.shape[1]), lambda i: (i, 0)),
        out_shape=jax.ShapeDtypeStruct((m, w.shape[1]), _f32),
    )(a, w, s)


def _add_mm_scale_body(r_ref, u_ref, w_ref, s_ref, o_ref):
    o_ref[...] = s_ref[...] * jnp.dot(r_ref[...] + u_ref[...], w_ref[...],
                                      preferred_element_type=_f32)


def _add_mm_scale(r, u, w, s):
    m, k = r.shape
    br = m // GRID_N
    return pl.pallas_call(
        _add_mm_scale_body,
        grid=(GRID_N,),
        in_specs=[
            pl.BlockSpec((br, k), lambda i: (i, 0)),
            pl.BlockSpec((br, k), lambda i: (i, 0)),
            pl.BlockSpec((k, w.shape[1]), lambda i: (0, 0)),
            pl.BlockSpec((br, 1), lambda i: (i, 0)),
        ],
        out_specs=pl.BlockSpec((br, w.shape[1]), lambda i: (i, 0)),
        out_shape=jax.ShapeDtypeStruct((m, w.shape[1]), _f32),
    )(r, u, w, s)


def _gcn_post_score_body(acc_ref, xs_ref, d_ref, b_ref, p_ref, x1_ref, sc_ref):
    x1 = jnp.maximum(d_ref[...] * (acc_ref[...] + xs_ref[...]) + b_ref[...],
                     0.0)
    x1_ref[...] = x1
    sc_ref[...] = jnp.tanh(jnp.dot(x1, p_ref[...],
                                   preferred_element_type=_f32))


def _gcn_post_score(acc, xs, dinv, b, p_hat):
    return pl.pallas_call(
        _gcn_post_score_body,
        grid=(GRID_N,),
        in_specs=[
            pl.BlockSpec((BR, H), lambda i: (i, 0)),
            pl.BlockSpec((BR, H), lambda i: (i, 0)),
            pl.BlockSpec((BR, 1), lambda i: (i, 0)),
            pl.BlockSpec((1, H), lambda i: (0, 0)),
            pl.BlockSpec((H, 1), lambda i: (0, 0)),
        ],
        out_specs=[
            pl.BlockSpec((BR, H), lambda i: (i, 0)),
            pl.BlockSpec((BR, 1), lambda i: (i, 0)),
        ],
        out_shape=[
            jax.ShapeDtypeStruct((N, H), _f32),
            jax.ShapeDtypeStruct((N, 1), _f32),
        ],
    )(acc, xs, dinv, b, p_hat)


def _pool_post_body(z_ref, u1_ref, u2_ref, d2_ref, dp_ref, b_ref, o_ref):
    t = (2.0 - d2_ref[...]) * z_ref[...] + 2.0 * u1_ref[...] + u2_ref[...]
    o_ref[...] = jnp.maximum(dp_ref[...] * t + b_ref[...], 0.0)


def _pool_post(z, u1p, u2p, d2p, dinvp, b):
    return pl.pallas_call(
        _pool_post_body,
        grid=(GRID_K,),
        in_specs=[
            pl.BlockSpec((BR, H), lambda i: (i, 0)),
            pl.BlockSpec((BR, H), lambda i: (i, 0)),
            pl.BlockSpec((BR, H), lambda i: (i, 0)),
            pl.BlockSpec((BR, 1), lambda i: (i, 0)),
            pl.BlockSpec((BR, 1), lambda i: (i, 0)),
            pl.BlockSpec((1, H), lambda i: (0, 0)),
        ],
        out_specs=pl.BlockSpec((BR, H), lambda i: (i, 0)),
        out_shape=jax.ShapeDtypeStruct((KP, H), _f32),
    )(z, u1p, u2p, d2p, dinvp, b)


def _gcn_post_bn_body(acc_ref, xs_ref, d_ref, b_ref, g2_ref, b2_ref, o_ref):
    h = jnp.maximum(d_ref[...] * (acc_ref[...] + xs_ref[...]) + b_ref[...],
                    0.0)
    o_ref[...] = h * g2_ref[...] + b2_ref[...]


def _gcn_post_bn(acc, xs, dinv, b, g2, b2):
    return pl.pallas_call(
        _gcn_post_bn_body,
        grid=(GRID_N,),
        in_specs=[
            pl.BlockSpec((BR, H), lambda i: (i, 0)),
            pl.BlockSpec((BR, H), lambda i: (i, 0)),
            pl.BlockSpec((BR, 1), lambda i: (i, 0)),
            pl.BlockSpec((1, H), lambda i: (0, 0)),
            pl.BlockSpec((1, H), lambda i: (0, 0)),
            pl.BlockSpec((1, H), lambda i: (0, 0)),
        ],
        out_specs=pl.BlockSpec((BR, H), lambda i: (i, 0)),
        out_shape=jax.ShapeDtypeStruct((N, H), _f32),
    )(acc, xs, dinv, b, g2, b2)


def _final_body(a_ref, w_ref, b_ref, o_ref):
    o_ref[...] = jnp.dot(a_ref[...], w_ref[...],
                         preferred_element_type=_f32) + b_ref[...]


def _final_linear(a, w, b):
    return pl.pallas_call(
        _final_body,
        grid=(GRID_N,),
        in_specs=[
            pl.BlockSpec((BR, H), lambda i: (i, 0)),
            pl.BlockSpec((H, w.shape[1]), lambda i: (0, 0)),
            pl.BlockSpec((1, w.shape[1]), lambda i: (0, 0)),
        ],
        out_specs=pl.BlockSpec((BR, w.shape[1]), lambda i: (i, 0)),
        out_shape=jax.ShapeDtypeStruct((N, w.shape[1]), _f32),
    )(a, w, b)


NTILES = 16
NCORES = 2
EPT = 5120
NCHUNK = 40
CHUNK = 128
E_PAD = NCORES * NTILES * EPT
N_PAD = 10240
ROWS_PER_TILE = N_PAD // NTILES


def _make_sc_edge_apply(width):
    lanes = 16
    nvec = width // lanes
    mesh = plsc.VectorSubcoreMesh(core_axis_name="c", subcore_axis_name="s")

    @functools.partial(
        pl.kernel,
        mesh=mesh,
        compiler_params=pltpu.CompilerParams(use_tc_tiling_on_sc=False),
        out_type=jax.ShapeDtypeStruct((NCORES, N_PAD, width), _f32),
        scratch_types=[
            pltpu.VMEM((CHUNK,), jnp.int32),
            pltpu.VMEM((CHUNK,), jnp.int32),
            pltpu.VMEM((CHUNK, lanes), _f32),
            pltpu.VMEM((CHUNK, width), _f32),
            pltpu.VMEM_SHARED((N_PAD, width), _f32),
            pltpu.SemaphoreType.DMA,
        ],
    )
    def apply_k(x_hbm, er_hbm, ec_hbm, ew_hbm, z_hbm, out_hbm,
                ridx, cidx, wv, rows, acc, sem):
        c = lax.axis_index("c")
        s = lax.axis_index("s")
        rs = pl.ds(s * ROWS_PER_TILE, ROWS_PER_TILE)
        pltpu.sync_copy(z_hbm.at[rs], acc.at[rs])
        plsc.subcore_barrier()

        def chunk(j, carry):
            pltpu.sync_copy(er_hbm.at[c, s, j], ridx)
            pltpu.sync_copy(ec_hbm.at[c, s, j], cidx)
            pltpu.sync_copy(ew_hbm.at[c, s, j], wv)
            pltpu.async_copy(x_hbm.at[ridx], rows, sem).wait()

            def scale16(g, cc):
                base = g * 16
                for l in range(16):
                    wrow = wv[base + l, :]
                    for f in range(nvec):
                        sl = pl.ds(f * lanes, lanes)
                        rows[base + l, sl] = rows[base + l, sl] * wrow
                return cc

            lax.fori_loop(0, CHUNK // 16, scale16, 0)
            pltpu.sync_copy(rows, acc.at[cidx], add=True)
            return carry

        lax.fori_loop(0, NCHUNK, chunk, 0)
        plsc.subcore_barrier()
        pltpu.sync_copy(acc.at[rs], out_hbm.at[c, rs])

    return apply_k


_sc_apply_64 = _make_sc_edge_apply(64)
_sc_apply_16 = _make_sc_edge_apply(16)


KP_PAD = 5120
RG_PER_TILE = KP_PAD // (NCORES * NTILES)
RG_CHUNK = 80


def _make_sc_row_gather(width):
    mesh = plsc.VectorSubcoreMesh(core_axis_name="c", subcore_axis_name="s")

    @functools.partial(
        pl.kernel,
        mesh=mesh,
        compiler_params=pltpu.CompilerParams(use_tc_tiling_on_sc=False),
        out_type=jax.ShapeDtypeStruct((KP_PAD, width), _f32),
        scratch_types=[
            pltpu.VMEM((2, RG_CHUNK), jnp.int32),
            pltpu.VMEM((RG_CHUNK, width), _f32),
            pltpu.SemaphoreType.DMA,
        ],
    )
    def gather_k(x_hbm, idx_hbm, out_hbm, idxv, rows, sem):
        c = lax.axis_index("c")
        s = lax.axis_index("s")
        tid = c * NTILES + s
        pltpu.sync_copy(idx_hbm.at[tid], idxv)
        for j in range(2):
            pltpu.async_copy(x_hbm.at[idxv.at[j]], rows, sem).wait()
            pltpu.sync_copy(
                rows, out_hbm.at[pl.ds(tid * RG_PER_TILE + j * RG_CHUNK,
                                       RG_CHUNK)])

    return gather_k


_sc_gather_64 = _make_sc_row_gather(64)


def _rows_gather(X, perm_pad):
    Xp = jnp.pad(X, ((0, N_PAD - N), (0, 0)))
    return _sc_gather_64(Xp, perm_pad)[:KP]


def _make_sc_row_scatter(width):
    mesh = plsc.VectorSubcoreMesh(core_axis_name="c", subcore_axis_name="s")

    @functools.partial(
        pl.kernel,
        mesh=mesh,
        compiler_params=pltpu.CompilerParams(use_tc_tiling_on_sc=False),
        out_type=jax.ShapeDtypeStruct((NCORES, N_PAD, width), _f32),
        scratch_types=[
            pltpu.VMEM((2, RG_CHUNK), jnp.int32),
            pltpu.VMEM((RG_CHUNK, width), _f32),
            pltpu.VMEM_SHARED((N_PAD, width), _f32),
            pltpu.SemaphoreType.DMA,
        ],
    )
    def scatter_k(z_hbm, idx_hbm, zero_hbm, out_hbm, idxv, rows, acc, sem):
        c = lax.axis_index("c")
        s = lax.axis_index("s")
        tid = c * NTILES + s
        rs = pl.ds(s * ROWS_PER_TILE, ROWS_PER_TILE)
        pltpu.sync_copy(zero_hbm.at[rs], acc.at[rs])
        pltpu.sync_copy(idx_hbm.at[tid], idxv)
        plsc.subcore_barrier()
        for j in range(2):
            base = tid * RG_PER_TILE + j * RG_CHUNK
            pltpu.sync_copy(z_hbm.at[pl.ds(base, RG_CHUNK)], rows)
            pltpu.sync_copy(rows, acc.at[idxv.at[j]], add=True)
        plsc.subcore_barrier()
        pltpu.sync_copy(acc.at[rs], out_hbm.at[c, rs])

    return scatter_k


_sc_scatter_64 = _make_sc_row_scatter(64)


def _rows_scatter(z, perm_pad, zeros64):
    parts = _sc_scatter_64(z, perm_pad, zeros64)
    return (parts[0] + parts[1])[:N]


def _prep_edges(row, col, w):
    E = row.shape[0]
    pad = E_PAD - E
    row_p = jnp.pad(row, (0, pad)).reshape(NCORES, NTILES, NCHUNK, CHUNK)
    col_p = jnp.pad(col, (0, pad)).reshape(NCORES, NTILES, NCHUNK, CHUNK)
    w_p = jnp.pad(w, (0, pad))
    w16 = jnp.broadcast_to(
        w_p[:, None], (E_PAD, 16)).reshape(NCORES, NTILES, NCHUNK, CHUNK, 16)
    return row_p, col_p, w16


def _edge_apply(X, ed, zeros64):
    Xp = jnp.pad(X, ((0, N_PAD - N), (0, 0)))
    parts = _sc_apply_64(Xp, ed[0], ed[1], ed[2], zeros64)
    return (parts[0] + parts[1])[:N]


def _edge_apply_16(X, ed, zeros16):
    Xp = jnp.pad(X, ((0, N_PAD - N), (0, 0)))
    parts = _sc_apply_16(Xp, ed[0], ed[1], ed[2], zeros16)
    return (parts[0] + parts[1])[:N]


def kernel(x, edge_weight, u1_W0, u1_b0, u1_p, u1_W1, u1_b1, u1_Wu, u1_bu,
           bn1_g, bn1_b, u2_W0, u2_b0, u2_p, u2_W1, u2_b1, u2_Wu, u2_bu,
           bn2_g, bn2_b, lin_W, lin_b, edge_index):
    n = x.shape[0]
    row = edge_index[0].astype(jnp.int32)
    col = edge_index[1].astype(jnp.int32)
    w = edge_weight
    E = w.shape[0]

    ed = _prep_edges(row, col, w)
    zeros64 = jnp.zeros((N_PAD, 64), _f32)
    zeros16 = jnp.zeros((N_PAD, 16), _f32)
    ones16 = jnp.ones((n, 16), _f32)

    deg = _edge_apply_16(ones16, ed, zeros16)[:, 0] + 1.0
    dinv = lax.rsqrt(deg)[:, None]

    key = row * n + col
    rkey = col * n + row
    K2 = jnp.concatenate([key, rkey])
    V2 = jnp.concatenate([w, jnp.zeros(E, _f32)])
    T2 = jnp.concatenate([jnp.full((E,), -1, jnp.int32),
                          jnp.arange(E, dtype=jnp.int32)])
    Ks, Vs, Ts = lax.sort((K2, V2, T2), num_keys=1)
    flags = jnp.concatenate([jnp.zeros(1, jnp.int32),
                             (Ks[1:] != Ks[:-1]).astype(jnp.int32)])
    runid = jnp.cumsum(flags)
    runsum = jnp.zeros(2 * E, _f32).at[runid].add(Vs)
    val = runsum[runid]
    isq = Ts >= 0
    revw = jnp.zeros(E, _f32).at[jnp.where(isq, Ts, 0)].add(
        jnp.where(isq, val, 0.0))
    wd2 = w * revw + 2.0 * jnp.where(row == col, w, 0.0)
    ed_d2 = _prep_edges(col, row, wd2)
    D2 = _edge_apply_16(ones16, ed_d2, zeros16)[:, 0] + 1.0

    def unet(xin, W0, b0, p, W1, b1, Wu, bu, g2, b2):
        xs = _mm_scale(xin, W0, dinv, GRID_N)
        acc = _edge_apply(xs, ed, zeros64)
        p_hat = (p / jnp.linalg.norm(p))[:, None]
        x1, score = _gcn_post_score(acc, xs, dinv, b0[None, :], p_hat)
        vals, perm = lax.top_k(score[:, 0], KP)
        perm_pad = jnp.pad(perm, (0, KP_PAD - KP)).reshape(
            NCORES * NTILES, 2, RG_CHUNK)

        mask = score[:, 0] >= vals[KP - 1]
        m16 = jnp.where(mask, 1.0, 0.0)[:, None] * jnp.ones((1, 16), _f32)
        Sm = _edge_apply_16(m16, ed, zeros16)
        SSm = _edge_apply_16(Sm, ed, zeros16)
        q = m16[:, 0] + 2.0 * Sm[:, 0] + SSm[:, 0]
        degp = q[perm] - D2[perm] + 1.0
        dinvp = jnp.where(degp > 0, lax.rsqrt(jnp.maximum(degp, 1e-12)),
                          0.0)[:, None]

        zscale = dinvp * vals[:, None]
        z = _mm_scale(_rows_gather(x1, perm_pad), W1, zscale, GRID_K)
        z_pad = jnp.pad(z, ((0, KP_PAD - KP), (0, 0)))
        Z = _rows_scatter(z_pad, perm_pad, zeros64)
        U1 = _edge_apply(Z, ed, zeros64)
        U2 = _edge_apply(U1, ed, zeros64)
        xp2 = _pool_post(z, _rows_gather(U1, perm_pad),
                         _rows_gather(U2, perm_pad), D2[perm][:, None],
                         dinvp, b1[None, :])
        UP = _rows_scatter(jnp.pad(xp2, ((0, KP_PAD - KP), (0, 0))),
                           perm_pad, zeros64)

        xs_u = _add_mm_scale(x1, UP, Wu, dinv)
        acc_u = _edge_apply(xs_u, ed, zeros64)
        return _gcn_post_bn(acc_u, xs_u, dinv, bu[None, :], g2[None, :],
                            b2[None, :])

    g2_1 = bn1_g / jnp.sqrt(1.0 + 1e-5)
    g2_2 = bn2_g / jnp.sqrt(1.0 + 1e-5)
    h = unet(x, u1_W0, u1_b0, u1_p, u1_W1, u1_b1, u1_Wu, u1_bu, g2_1, bn1_b)
    h = unet(h, u2_W0, u2_b0, u2_p, u2_W1, u2_b1, u2_Wu, u2_bu, g2_2, bn2_b)
    return _final_linear(h, lin_W, lin_b[None, :])


# speedup vs baseline: 3.7391x; 1.2366x over previous
import functools
import math

import jax
import jax.numpy as jnp
from jax import lax
from jax.experimental import pallas as pl
from jax.experimental.pallas import tpu as pltpu
from jax.experimental.pallas import tpu_sc as plsc

N = 10000
H = 64
BR = 1000
GRID_N = N // BR
KP = 5000
GRID_K = KP // BR

_f32 = jnp.float32


def _mm_scale_body(a_ref, w_ref, s_ref, o_ref):
    o_ref[...] = s_ref[...] * jnp.dot(a_ref[...], w_ref[...],
                                      preferred_element_type=_f32)


def _mm_scale(a, w, s, grid):
    m, k = a.shape
    br = m // grid
    return pl.pallas_call(
        _mm_scale_body,
        grid=(grid,),
        in_specs=[
            pl.BlockSpec((br, k), lambda i: (i, 0)),
            pl.BlockSpec((k, w.shape[1]), lambda i: (0, 0)),
            pl.BlockSpec((br, 1), lambda i: (i, 0)),
        ],
        out_specs=pl.BlockSpec((br, w.shape[1]), lambda i: (i, 0)),
        out_shape=jax.ShapeDtypeStruct((m, w.shape[1]), _f32),
    )(a, w, s)


def _add_mm_scale_body(r_ref, u_ref, w_ref, s_ref, o_ref):
    o_ref[...] = s_ref[...] * jnp.dot(r_ref[...] + u_ref[...], w_ref[...],
                                      preferred_element_type=_f32)


def _add_mm_scale(r, u, w, s):
    m, k = r.shape
    br = m // GRID_N
    return pl.pallas_call(
        _add_mm_scale_body,
        grid=(GRID_N,),
        in_specs=[
            pl.BlockSpec((br, k), lambda i: (i, 0)),
            pl.BlockSpec((br, k), lambda i: (i, 0)),
            pl.BlockSpec((k, w.shape[1]), lambda i: (0, 0)),
            pl.BlockSpec((br, 1), lambda i: (i, 0)),
        ],
        out_specs=pl.BlockSpec((br, w.shape[1]), lambda i: (i, 0)),
        out_shape=jax.ShapeDtypeStruct((m, w.shape[1]), _f32),
    )(r, u, w, s)


def _gcn_post_score_body(acc_ref, xs_ref, d_ref, b_ref, p_ref, x1_ref, sc_ref):
    x1 = jnp.maximum(d_ref[...] * (acc_ref[...] + xs_ref[...]) + b_ref[...],
                     0.0)
    x1_ref[...] = x1
    sc_ref[...] = jnp.tanh(jnp.dot(x1, p_ref[...],
                                   preferred_element_type=_f32))


def _gcn_post_score(acc, xs, dinv, b, p_hat):
    return pl.pallas_call(
        _gcn_post_score_body,
        grid=(GRID_N,),
        in_specs=[
            pl.BlockSpec((BR, H), lambda i: (i, 0)),
            pl.BlockSpec((BR, H), lambda i: (i, 0)),
            pl.BlockSpec((BR, 1), lambda i: (i, 0)),
            pl.BlockSpec((1, H), lambda i: (0, 0)),
            pl.BlockSpec((H, 1), lambda i: (0, 0)),
        ],
        out_specs=[
            pl.BlockSpec((BR, H), lambda i: (i, 0)),
            pl.BlockSpec((BR, 1), lambda i: (i, 0)),
        ],
        out_shape=[
            jax.ShapeDtypeStruct((N, H), _f32),
            jax.ShapeDtypeStruct((N, 1), _f32),
        ],
    )(acc, xs, dinv, b, p_hat)


def _pool_post_body(z_ref, u1_ref, u2_ref, d2_ref, dp_ref, b_ref, o_ref):
    t = (2.0 - d2_ref[...]) * z_ref[...] + 2.0 * u1_ref[...] + u2_ref[...]
    o_ref[...] = jnp.maximum(dp_ref[...] * t + b_ref[...], 0.0)


def _pool_post(z, u1p, u2p, d2p, dinvp, b):
    return pl.pallas_call(
        _pool_post_body,
        grid=(GRID_K,),
        in_specs=[
            pl.BlockSpec((BR, H), lambda i: (i, 0)),
            pl.BlockSpec((BR, H), lambda i: (i, 0)),
            pl.BlockSpec((BR, H), lambda i: (i, 0)),
            pl.BlockSpec((BR, 1), lambda i: (i, 0)),
            pl.BlockSpec((BR, 1), lambda i: (i, 0)),
            pl.BlockSpec((1, H), lambda i: (0, 0)),
        ],
        out_specs=pl.BlockSpec((BR, H), lambda i: (i, 0)),
        out_shape=jax.ShapeDtypeStruct((KP, H), _f32),
    )(z, u1p, u2p, d2p, dinvp, b)


def _gcn_post_bn_body(acc_ref, xs_ref, d_ref, b_ref, g2_ref, b2_ref, o_ref):
    h = jnp.maximum(d_ref[...] * (acc_ref[...] + xs_ref[...]) + b_ref[...],
                    0.0)
    o_ref[...] = h * g2_ref[...] + b2_ref[...]


def _gcn_post_bn(acc, xs, dinv, b, g2, b2):
    return pl.pallas_call(
        _gcn_post_bn_body,
        grid=(GRID_N,),
        in_specs=[
            pl.BlockSpec((BR, H), lambda i: (i, 0)),
            pl.BlockSpec((BR, H), lambda i: (i, 0)),
            pl.BlockSpec((BR, 1), lambda i: (i, 0)),
            pl.BlockSpec((1, H), lambda i: (0, 0)),
            pl.BlockSpec((1, H), lambda i: (0, 0)),
            pl.BlockSpec((1, H), lambda i: (0, 0)),
        ],
        out_specs=pl.BlockSpec((BR, H), lambda i: (i, 0)),
        out_shape=jax.ShapeDtypeStruct((N, H), _f32),
    )(acc, xs, dinv, b, g2, b2)


def _final_body(a_ref, w_ref, b_ref, o_ref):
    o_ref[...] = jnp.dot(a_ref[...], w_ref[...],
                         preferred_element_type=_f32) + b_ref[...]


def _final_linear(a, w, b):
    return pl.pallas_call(
        _final_body,
        grid=(GRID_N,),
        in_specs=[
            pl.BlockSpec((BR, H), lambda i: (i, 0)),
            pl.BlockSpec((H, w.shape[1]), lambda i: (0, 0)),
            pl.BlockSpec((1, w.shape[1]), lambda i: (0, 0)),
        ],
        out_specs=pl.BlockSpec((BR, w.shape[1]), lambda i: (i, 0)),
        out_shape=jax.ShapeDtypeStruct((N, w.shape[1]), _f32),
    )(a, w, b)


NTILES = 16
NCORES = 2
EPT = 5120
NCHUNK = 40
CHUNK = 128
E_PAD = NCORES * NTILES * EPT
N_PAD = 10240
ROWS_PER_TILE = N_PAD // NTILES


def _make_sc_edge_apply(width):
    lanes = 16
    nvec = width // lanes
    mesh = plsc.VectorSubcoreMesh(core_axis_name="c", subcore_axis_name="s")

    @functools.partial(
        pl.kernel,
        mesh=mesh,
        compiler_params=pltpu.CompilerParams(use_tc_tiling_on_sc=False),
        out_type=jax.ShapeDtypeStruct((NCORES, N_PAD, width), _f32),
        scratch_types=[
            pltpu.VMEM((NCHUNK, CHUNK), jnp.int32),
            pltpu.VMEM((NCHUNK, CHUNK), jnp.int32),
            pltpu.VMEM((CHUNK, lanes), _f32),
            pltpu.VMEM((CHUNK, lanes), _f32),
            pltpu.VMEM((CHUNK, width), _f32),
            pltpu.VMEM((CHUNK, width), _f32),
            pltpu.VMEM_SHARED((N_PAD, width), _f32),
            pltpu.SemaphoreType.DMA,
            pltpu.SemaphoreType.DMA,
            pltpu.SemaphoreType.DMA,
            pltpu.SemaphoreType.DMA,
        ],
    )
    def apply_k(x_hbm, er_hbm, ec_hbm, ew_hbm, z_hbm, out_hbm,
                ridx_all, cidx_all, wv_a, wv_b, rows_a, rows_b, acc,
                sem_ga, sem_gb, sem_wa, sem_wb):
        c = lax.axis_index("c")
        s = lax.axis_index("s")
        rs = pl.ds(s * ROWS_PER_TILE, ROWS_PER_TILE)
        pltpu.sync_copy(z_hbm.at[rs], acc.at[rs])
        pltpu.sync_copy(er_hbm.at[c, s], ridx_all)
        pltpu.sync_copy(ec_hbm.at[c, s], cidx_all)
        plsc.subcore_barrier()

        bufs = [(rows_a, wv_a, sem_ga, sem_wa),
                (rows_b, wv_b, sem_gb, sem_wb)]
        for b in range(2):
            rows, wv, sg, sw_ = bufs[b]
            pltpu.async_copy(x_hbm.at[ridx_all.at[b]], rows, sg)
            pltpu.async_copy(ew_hbm.at[c, s, b], wv, sw_)

        def outer(it, carry):
            for b in range(2):
                j = it * 2 + b
                rows, wv, sg, sw_ = bufs[b]
                pltpu.make_async_copy(x_hbm.at[ridx_all.at[j]], rows,
                                      sg).wait()
                pltpu.make_async_copy(ew_hbm.at[c, s, j], wv, sw_).wait()

                def scale16(g, cc):
                    base = g * 16
                    for l in range(16):
                        wrow = wv[base + l, :]
                        for f in range(nvec):
                            sl = pl.ds(f * lanes, lanes)
                            rows[base + l, sl] = rows[base + l, sl] * wrow
                    return cc

                lax.fori_loop(0, CHUNK // 16, scale16, 0)
                pltpu.sync_copy(rows, acc.at[cidx_all.at[j]], add=True)

                @pl.when(j + 2 < NCHUNK)
                def _():
                    pltpu.async_copy(x_hbm.at[ridx_all.at[j + 2]], rows, sg)
                    pltpu.async_copy(ew_hbm.at[c, s, j + 2], wv, sw_)
            return carry

        lax.fori_loop(0, NCHUNK // 2, outer, 0)
        plsc.subcore_barrier()
        pltpu.sync_copy(acc.at[rs], out_hbm.at[c, rs])

    return apply_k


_sc_apply_64 = _make_sc_edge_apply(64)
_sc_apply_16 = _make_sc_edge_apply(16)


KP_PAD = 5120
RG_PER_TILE = KP_PAD // (NCORES * NTILES)
RG_CHUNK = 80


def _make_sc_row_gather(width):
    mesh = plsc.VectorSubcoreMesh(core_axis_name="c", subcore_axis_name="s")

    @functools.partial(
        pl.kernel,
        mesh=mesh,
        compiler_params=pltpu.CompilerParams(use_tc_tiling_on_sc=False),
        out_type=jax.ShapeDtypeStruct((KP_PAD, width), _f32),
        scratch_types=[
            pltpu.VMEM((2, RG_CHUNK), jnp.int32),
            pltpu.VMEM((RG_CHUNK, width), _f32),
            pltpu.SemaphoreType.DMA,
        ],
    )
    def gather_k(x_hbm, idx_hbm, out_hbm, idxv, rows, sem):
        c = lax.axis_index("c")
        s = lax.axis_index("s")
        tid = c * NTILES + s
        pltpu.sync_copy(idx_hbm.at[tid], idxv)
        for j in range(2):
            pltpu.async_copy(x_hbm.at[idxv.at[j]], rows, sem).wait()
            pltpu.sync_copy(
                rows, out_hbm.at[pl.ds(tid * RG_PER_TILE + j * RG_CHUNK,
                                       RG_CHUNK)])

    return gather_k


_sc_gather_64 = _make_sc_row_gather(64)


def _rows_gather(X, perm_pad):
    Xp = jnp.pad(X, ((0, N_PAD - N), (0, 0)))
    return _sc_gather_64(Xp, perm_pad)[:KP]


def _make_sc_row_scatter(width):
    mesh = plsc.VectorSubcoreMesh(core_axis_name="c", subcore_axis_name="s")

    @functools.partial(
        pl.kernel,
        mesh=mesh,
        compiler_params=pltpu.CompilerParams(use_tc_tiling_on_sc=False),
        out_type=jax.ShapeDtypeStruct((NCORES, N_PAD, width), _f32),
        scratch_types=[
            pltpu.VMEM((2, RG_CHUNK), jnp.int32),
            pltpu.VMEM((RG_CHUNK, width), _f32),
            pltpu.VMEM_SHARED((N_PAD, width), _f32),
            pltpu.SemaphoreType.DMA,
        ],
    )
    def scatter_k(z_hbm, idx_hbm, zero_hbm, out_hbm, idxv, rows, acc, sem):
        c = lax.axis_index("c")
        s = lax.axis_index("s")
        tid = c * NTILES + s
        rs = pl.ds(s * ROWS_PER_TILE, ROWS_PER_TILE)
        pltpu.sync_copy(zero_hbm.at[rs], acc.at[rs])
        pltpu.sync_copy(idx_hbm.at[tid], idxv)
        plsc.subcore_barrier()
        for j in range(2):
            base = tid * RG_PER_TILE + j * RG_CHUNK
            pltpu.sync_copy(z_hbm.at[pl.ds(base, RG_CHUNK)], rows)
            pltpu.sync_copy(rows, acc.at[idxv.at[j]], add=True)
        plsc.subcore_barrier()
        pltpu.sync_copy(acc.at[rs], out_hbm.at[c, rs])

    return scatter_k


_sc_scatter_64 = _make_sc_row_scatter(64)


def _rows_scatter(z, perm_pad, zeros64):
    parts = _sc_scatter_64(z, perm_pad, zeros64)
    return (parts[0] + parts[1])[:N]


def _prep_edges(row, col, w):
    E = row.shape[0]
    pad = E_PAD - E
    row_p = jnp.pad(row, (0, pad)).reshape(NCORES, NTILES, NCHUNK, CHUNK)
    col_p = jnp.pad(col, (0, pad)).reshape(NCORES, NTILES, NCHUNK, CHUNK)
    w_p = jnp.pad(w, (0, pad))
    w16 = jnp.broadcast_to(
        w_p[:, None], (E_PAD, 16)).reshape(NCORES, NTILES, NCHUNK, CHUNK, 16)
    return row_p, col_p, w16


def _edge_apply(X, ed, zeros64):
    Xp = jnp.pad(X, ((0, N_PAD - N), (0, 0)))
    parts = _sc_apply_64(Xp, ed[0], ed[1], ed[2], zeros64)
    return (parts[0] + parts[1])[:N]


def _edge_apply_16(X, ed, zeros16):
    Xp = jnp.pad(X, ((0, N_PAD - N), (0, 0)))
    parts = _sc_apply_16(Xp, ed[0], ed[1], ed[2], zeros16)
    return (parts[0] + parts[1])[:N]


def kernel(x, edge_weight, u1_W0, u1_b0, u1_p, u1_W1, u1_b1, u1_Wu, u1_bu,
           bn1_g, bn1_b, u2_W0, u2_b0, u2_p, u2_W1, u2_b1, u2_Wu, u2_bu,
           bn2_g, bn2_b, lin_W, lin_b, edge_index):
    n = x.shape[0]
    row = edge_index[0].astype(jnp.int32)
    col = edge_index[1].astype(jnp.int32)
    w = edge_weight
    E = w.shape[0]

    ed = _prep_edges(row, col, w)
    zeros64 = jnp.zeros((N_PAD, 64), _f32)
    zeros16 = jnp.zeros((N_PAD, 16), _f32)
    ones16 = jnp.ones((n, 16), _f32)

    deg = _edge_apply_16(ones16, ed, zeros16)[:, 0] + 1.0
    dinv = lax.rsqrt(deg)[:, None]

    key = row * n + col
    rkey = col * n + row
    K2 = jnp.concatenate([key, rkey])
    V2 = jnp.concatenate([w, jnp.zeros(E, _f32)])
    T2 = jnp.concatenate([jnp.full((E,), -1, jnp.int32),
                          jnp.arange(E, dtype=jnp.int32)])
    Ks, Vs, Ts = lax.sort((K2, V2, T2), num_keys=1)
    flags = jnp.concatenate([jnp.zeros(1, jnp.int32),
                             (Ks[1:] != Ks[:-1]).astype(jnp.int32)])
    runid = jnp.cumsum(flags)
    runsum = jnp.zeros(2 * E, _f32).at[runid].add(Vs)
    val = runsum[runid]
    isq = Ts >= 0
    revw = jnp.zeros(E, _f32).at[jnp.where(isq, Ts, 0)].add(
        jnp.where(isq, val, 0.0))
    wd2 = w * revw + 2.0 * jnp.where(row == col, w, 0.0)
    ed_d2 = _prep_edges(col, row, wd2)
    D2 = _edge_apply_16(ones16, ed_d2, zeros16)[:, 0] + 1.0

    def unet(xin, W0, b0, p, W1, b1, Wu, bu, g2, b2):
        xs = _mm_scale(xin, W0, dinv, GRID_N)
        acc = _edge_apply(xs, ed, zeros64)
        p_hat = (p / jnp.linalg.norm(p))[:, None]
        x1, score = _gcn_post_score(acc, xs, dinv, b0[None, :], p_hat)
        vals, perm = lax.top_k(score[:, 0], KP)
        perm_pad = jnp.pad(perm, (0, KP_PAD - KP)).reshape(
            NCORES * NTILES, 2, RG_CHUNK)

        mask = score[:, 0] >= vals[KP - 1]
        m16 = jnp.where(mask, 1.0, 0.0)[:, None] * jnp.ones((1, 16), _f32)
        Sm = _edge_apply_16(m16, ed, zeros16)
        SSm = _edge_apply_16(Sm, ed, zeros16)
        q = m16[:, 0] + 2.0 * Sm[:, 0] + SSm[:, 0]
        degp = q[perm] - D2[perm] + 1.0
        dinvp = jnp.where(degp > 0, lax.rsqrt(jnp.maximum(degp, 1e-12)),
                          0.0)[:, None]

        zscale = dinvp * vals[:, None]
        z = _mm_scale(_rows_gather(x1, perm_pad), W1, zscale, GRID_K)
        z_pad = jnp.pad(z, ((0, KP_PAD - KP), (0, 0)))
        Z = _rows_scatter(z_pad, perm_pad, zeros64)
        U1 = _edge_apply(Z, ed, zeros64)
        U2 = _edge_apply(U1, ed, zeros64)
        xp2 = _pool_post(z, _rows_gather(U1, perm_pad),
                         _rows_gather(U2, perm_pad), D2[perm][:, None],
                         dinvp, b1[None, :])
        UP = _rows_scatter(jnp.pad(xp2, ((0, KP_PAD - KP), (0, 0))),
                           perm_pad, zeros64)

        xs_u = _add_mm_scale(x1, UP, Wu, dinv)
        acc_u = _edge_apply(xs_u, ed, zeros64)
        return _gcn_post_bn(acc_u, xs_u, dinv, bu[None, :], g2[None, :],
                            b2[None, :])

    g2_1 = bn1_g / jnp.sqrt(1.0 + 1e-5)
    g2_2 = bn2_g / jnp.sqrt(1.0 + 1e-5)
    h = unet(x, u1_W0, u1_b0, u1_p, u1_W1, u1_b1, u1_Wu, u1_bu, g2_1, bn1_b)
    h = unet(h, u2_W0, u2_b0, u2_p, u2_W1, u2_b1, u2_Wu, u2_bu, g2_2, bn2_b)
    return _final_linear(h, lin_W, lin_b[None, :])
